# baseline (speedup 1.0000x reference)
import functools

import jax
import jax.numpy as jnp
from jax import lax
from jax.experimental import pallas as pl
from jax.experimental.pallas import tpu as pltpu
from jax.experimental.pallas import tpu_sc as plsc

N = 10000
E = 320000
D = 128
DH = D // 2

NC = 2
NS = 16
NW = NC * NS
L = 16

KB = 80
EPT = E // NW
NB = EPT // KB

CHUNK = 640
LAST = N - CHUNK * (NS - 1)
N_PAD = CHUNK * NS

_mesh = plsc.VectorSubcoreMesh(core_axis_name="c", subcore_axis_name="s")


@functools.partial(
    pl.kernel,
    out_type=jax.ShapeDtypeStruct((NC, 1, N_PAD), jnp.float32),
    mesh=_mesh,
    scratch_types=[
        pltpu.VMEM((NB, KB), jnp.int32),
        pltpu.VMEM((NB, KB), jnp.float32),
        pltpu.VMEM((CHUNK,), jnp.float32),
        pltpu.VMEM_SHARED((N_PAD,), jnp.float32),
    ],
)
def _deg_kernel(col_hbm, attr_hbm, out_hbm, col_v, attr_v, fill_v, deg_sh):
    c = lax.axis_index("c")
    s = lax.axis_index("s")
    tile = c * NS + s

    pltpu.sync_copy(col_hbm.at[tile], col_v)
    pltpu.sync_copy(attr_hbm.at[tile], attr_v)

    def fill(i, _):
        fill_v[pl.ds(i * L, L)] = jnp.full((L,), 0.5, jnp.float32)
        return 0
    lax.fori_loop(0, CHUNK // L, fill, 0)
    pltpu.sync_copy(fill_v, deg_sh.at[pl.ds(s * CHUNK, CHUNK)])

    plsc.subcore_barrier()

    def acc(b, _):
        pltpu.sync_copy(attr_v.at[b], deg_sh.at[col_v.at[b]], add=True)
        return 0
    lax.fori_loop(0, NB, acc, 0)

    plsc.subcore_barrier()

    pltpu.sync_copy(deg_sh.at[pl.ds(s * CHUNK, CHUNK)],
                    out_hbm.at[c, 0, pl.ds(s * CHUNK, CHUNK)])


def _mm_body(x_ref, w_ref, y_ref):
    y_ref[0] = jnp.dot(x_ref[...], w_ref[0],
                       preferred_element_type=jnp.float32)


def _matmul_halves(x, W_split):
    blk = 1000
    return pl.pallas_call(
        _mm_body,
        grid=(N // blk, 2),
        in_specs=[
            pl.BlockSpec((blk, D), lambda i, h: (i, 0)),
            pl.BlockSpec((1, D, DH), lambda i, h: (h, 0, 0)),
        ],
        out_specs=pl.BlockSpec((1, blk, DH), lambda i, h: (h, i, 0)),
        out_shape=jax.ShapeDtypeStruct((2, N, DH), jnp.float32),
    )(x, W_split)


def _dis_body(degt_ref, dis_ref):
    deg = degt_ref[:, 0:1] + degt_ref[:, 1:2]
    dis_ref[...] = jnp.where(deg > 0, lax.rsqrt(deg), 0.0)


def _dis_kernel(deg_t):
    blk = 1000
    return pl.pallas_call(
        _dis_body,
        grid=(N // blk,),
        in_specs=[pl.BlockSpec((blk, NC), lambda i: (i, 0))],
        out_specs=pl.BlockSpec((blk, 1), lambda i: (i, 0)),
        out_shape=jax.ShapeDtypeStruct((N, 1), jnp.float32),
    )(deg_t)


@functools.partial(
    pl.kernel,
    out_type=jax.ShapeDtypeStruct((NC, 2, N, DH), jnp.float32),
    mesh=_mesh,
    scratch_types=[
        pltpu.VMEM((NB, KB), jnp.int32),
        pltpu.VMEM((NB, KB), jnp.int32),
        pltpu.VMEM((NB, KB), jnp.float32),
        pltpu.VMEM((N,), jnp.float32),
        pltpu.VMEM((KB, DH), jnp.float32),
        pltpu.VMEM((KB, DH), jnp.float32),
        pltpu.VMEM((KB, DH), jnp.float32),
        pltpu.VMEM((KB, DH), jnp.float32),
        pltpu.VMEM((KB, DH), jnp.float32),
        pltpu.VMEM((KB, DH), jnp.float32),
        pltpu.VMEM_SHARED((N, DH), jnp.float32),
        pltpu.SemaphoreType.DMA,
        pltpu.SemaphoreType.DMA,
        pltpu.SemaphoreType.DMA,
        pltpu.SemaphoreType.DMA,
        pltpu.SemaphoreType.DMA,
        pltpu.SemaphoreType.DMA,
    ],
    compiler_params=pltpu.CompilerParams(use_tc_tiling_on_sc=False,
                                         needs_layout_passes=False),
)
def _agg_kernel(y_hbm, dis_hbm, row_hbm, col_hbm, attr_hbm, out_hbm,
                row_v, col_v, attr_v, dis_v, g0, g1, g2, g3, s0, s1,
                acc_sh, gs0, gs1, gs2, gs3, ss0, ss1):
    c = lax.axis_index("c")
    s = lax.axis_index("s")
    tile = c * NS + s

    pltpu.sync_copy(row_hbm.at[tile], row_v)
    pltpu.sync_copy(col_hbm.at[tile], col_v)
    pltpu.sync_copy(attr_hbm.at[tile], attr_v)
    pltpu.sync_copy(dis_hbm, dis_v)

    gbufs = (g0, g1, g2, g3)
    gsems = (gs0, gs1, gs2, gs3)
    sbufs = (s0, s1)
    ssems = (ss0, ss1)

    for h in range(2):
        y_half = y_hbm.at[h]

        def zfill(i, _):
            for j in range(DH // L):
                g0[i, pl.ds(j * L, L)] = jnp.zeros((L,), jnp.float32)
            return 0
        lax.fori_loop(0, KB, zfill, 0)

        @pl.when(s < NS - 1)
        def _():
            def zput(i, _):
                pltpu.sync_copy(g0,
                                acc_sh.at[pl.ds(s * CHUNK + i * KB, KB), :])
                return 0
            lax.fori_loop(0, CHUNK // KB, zput, 0)

        @pl.when(s == NS - 1)
        def _():
            def zput(i, _):
                pltpu.sync_copy(
                    g0, acc_sh.at[pl.ds((NS - 1) * CHUNK + i * KB, KB), :])
                return 0
            lax.fori_loop(0, LAST // KB, zput, 0)

        plsc.subcore_barrier()

        def gather_start(b, buf, sem):
            pltpu.make_async_copy(y_half.at[row_v.at[b]], buf, sem).start()

        def gather_wait(b, buf, sem):
            pltpu.make_async_copy(y_half.at[row_v.at[b]], buf, sem).wait()

        def escale(b, gbuf, sbuf):
            def esc(g, _):
                idxv = row_v[b, pl.ds(g * L, L)]
                wvec = plsc.load_gather(dis_v, [idxv]) \
                    * attr_v[b, pl.ds(g * L, L)]
                for k in range(L):
                    w = wvec[k]
                    e = g * L + k
                    for j in range(DH // L):
                        sbuf[e, pl.ds(j * L, L)] = gbuf[e, pl.ds(j * L, L)] * w
                return 0
            lax.fori_loop(0, KB // L, esc, 0)

        def scatter_start(b, sbuf, sem):
            pltpu.async_copy(sbuf, acc_sh.at[col_v.at[b]], sem, add=True)

        def scatter_wait(b, sbuf, sem):
            pltpu.make_async_copy(sbuf, acc_sh.at[col_v.at[b]], sem).wait()

        for x in range(4):
            gather_start(x, gbufs[x], gsems[x])

        def pipe(j, _):
            for x in range(4):
                b = 4 * j + x
                gather_wait(b, gbufs[x], gsems[x])
                if x >= 2:
                    scatter_wait(b - 2, sbufs[x % 2], ssems[x % 2])
                else:
                    @pl.when(j > 0)
                    def _():
                        scatter_wait(b - 2, sbufs[x % 2], ssems[x % 2])
                escale(b, gbufs[x], sbufs[x % 2])

                @pl.when(b + 4 <= NB - 1)
                def _():
                    gather_start(b + 4, gbufs[x], gsems[x])
                scatter_start(b, sbufs[x % 2], ssems[x % 2])
            return 0
        lax.fori_loop(0, (NB - 1) // 4, pipe, 0)

        gather_wait(NB - 1, g0, gs0)
        scatter_wait(NB - 3, s0, ss0)
        escale(NB - 1, g0, s0)
        scatter_start(NB - 1, s0, ss0)
        scatter_wait(NB - 2, s1, ss1)
        scatter_wait(NB - 1, s0, ss0)

        plsc.subcore_barrier()

        @pl.when(s < NS - 1)
        def _():
            pltpu.sync_copy(acc_sh.at[pl.ds(s * CHUNK, CHUNK), :],
                            out_hbm.at[c, h, pl.ds(s * CHUNK, CHUNK), :])

        @pl.when(s == NS - 1)
        def _():
            pltpu.sync_copy(acc_sh.at[pl.ds((NS - 1) * CHUNK, LAST), :],
                            out_hbm.at[c, h, pl.ds((NS - 1) * CHUNK, LAST), :])


def _ep_body(x_ref, y_ref, p_ref, degt_ref, b_ref, out_ref):
    deg = degt_ref[:, 0:1] + degt_ref[:, 1:2]
    dis = jnp.where(deg > 0, lax.rsqrt(deg), 0.0)
    y_full = jnp.concatenate([y_ref[0], y_ref[1]], axis=1)
    p_full = jnp.concatenate(
        [p_ref[0, 0] + p_ref[1, 0], p_ref[0, 1] + p_ref[1, 1]], axis=1)
    agg = (p_full + y_full * dis) * dis + b_ref[...]
    out_ref[...] = x_ref[...] + jnp.maximum(agg, 0.0)


def _epilogue(x, y, p, deg_t, b):
    blk = 1000
    return pl.pallas_call(
        _ep_body,
        grid=(N // blk,),
        in_specs=[
            pl.BlockSpec((blk, D), lambda i: (i, 0)),
            pl.BlockSpec((2, blk, DH), lambda i: (0, i, 0)),
            pl.BlockSpec((NC, 2, blk, DH), lambda i: (0, 0, i, 0)),
            pl.BlockSpec((blk, NC), lambda i: (i, 0)),
            pl.BlockSpec((1, D), lambda i: (0, 0)),
        ],
        out_specs=pl.BlockSpec((blk, D), lambda i: (i, 0)),
        out_shape=jax.ShapeDtypeStruct((N, D), jnp.float32),
    )(x, y, p, deg_t, b)


def kernel(x, edge_index, edge_attr, W, b):
    row3d = edge_index[0].astype(jnp.int32).reshape(NW, NB, KB)
    col3d = edge_index[1].astype(jnp.int32).reshape(NW, NB, KB)
    attr3d = edge_attr.reshape(NW, NB, KB)

    w_split = jnp.transpose(W.reshape(D, 2, DH), (1, 0, 2))
    xw = _matmul_halves(x, w_split)
    deg_part = _deg_kernel(col3d, attr3d)
    deg_t = jnp.transpose(deg_part.reshape(NC, N_PAD)[:, :N])
    dis = _dis_kernel(deg_t).reshape(N)
    p = _agg_kernel(xw, dis, row3d, col3d, attr3d)
    return _epilogue(x, xw, p, deg_t, b.reshape(1, D))


# speedup vs baseline: 2.3775x; 2.3775x over previous
import functools

import jax
import jax.numpy as jnp
from jax import lax
from jax.experimental import pallas as pl
from jax.experimental.pallas import tpu as pltpu
from jax.experimental.pallas import tpu_sc as plsc

N = 10000
E = 320000
D = 128
DH = D // 2

NC = 2
NS = 16
NW = NC * NS
L = 16

KB = 80
EPT = E // NW
NB = EPT // KB

CHUNK = 640
LAST = N - CHUNK * (NS - 1)
N_PAD = CHUNK * NS

_mesh = plsc.VectorSubcoreMesh(core_axis_name="c", subcore_axis_name="s")


@functools.partial(
    pl.kernel,
    out_type=jax.ShapeDtypeStruct((NC, 1, N_PAD), jnp.float32),
    mesh=_mesh,
    scratch_types=[
        pltpu.VMEM((NB, KB), jnp.int32),
        pltpu.VMEM((NB, KB), jnp.float32),
        pltpu.VMEM((CHUNK,), jnp.float32),
        pltpu.VMEM_SHARED((N_PAD,), jnp.float32),
    ],
)
def _deg_kernel(col_hbm, attr_hbm, out_hbm, col_v, attr_v, fill_v, deg_sh):
    c = lax.axis_index("c")
    s = lax.axis_index("s")
    tile = c * NS + s

    pltpu.sync_copy(col_hbm.at[tile], col_v)
    pltpu.sync_copy(attr_hbm.at[tile], attr_v)

    def fill(i, _):
        fill_v[pl.ds(i * L, L)] = jnp.full((L,), 0.5, jnp.float32)
        return 0
    lax.fori_loop(0, CHUNK // L, fill, 0)
    pltpu.sync_copy(fill_v, deg_sh.at[pl.ds(s * CHUNK, CHUNK)])

    plsc.subcore_barrier()

    def acc(b, _):
        pltpu.sync_copy(attr_v.at[b], deg_sh.at[col_v.at[b]], add=True)
        return 0
    lax.fori_loop(0, NB, acc, 0)

    plsc.subcore_barrier()

    pltpu.sync_copy(deg_sh.at[pl.ds(s * CHUNK, CHUNK)],
                    out_hbm.at[c, 0, pl.ds(s * CHUNK, CHUNK)])


def _mm_body(x_ref, w_ref, degt_ref, y_ref):
    deg = degt_ref[:, 0:1] + degt_ref[:, 1:2]
    dis = jnp.where(deg > 0, lax.rsqrt(deg), 0.0)
    xw = jnp.dot(x_ref[...], w_ref[0], preferred_element_type=jnp.float32)
    y_ref[0] = xw * dis


def _matmul_scaled(x, W_split, deg_t):
    blk = 1000
    return pl.pallas_call(
        _mm_body,
        grid=(N // blk, 2),
        in_specs=[
            pl.BlockSpec((blk, D), lambda i, h: (i, 0)),
            pl.BlockSpec((1, D, DH), lambda i, h: (h, 0, 0)),
            pl.BlockSpec((blk, NC), lambda i, h: (i, 0)),
        ],
        out_specs=pl.BlockSpec((1, blk, DH), lambda i, h: (h, i, 0)),
        out_shape=jax.ShapeDtypeStruct((2, N, DH), jnp.float32),
    )(x, W_split, deg_t)


@functools.partial(
    pl.kernel,
    out_type=jax.ShapeDtypeStruct((NC, 2, N, DH), jnp.float32),
    mesh=_mesh,
    scratch_types=[
        pltpu.VMEM((NB, KB), jnp.int32),
        pltpu.VMEM((NB, KB), jnp.int32),
        pltpu.VMEM((NB, KB), jnp.float32),
        pltpu.VMEM((KB, DH), jnp.float32),
        pltpu.VMEM((KB, DH), jnp.float32),
        pltpu.VMEM((KB, DH), jnp.float32),
        pltpu.VMEM((KB, DH), jnp.float32),
        pltpu.VMEM((KB, DH), jnp.float32),
        pltpu.VMEM((KB, DH), jnp.float32),
        pltpu.VMEM_SHARED((N, DH), jnp.float32),
        pltpu.SemaphoreType.DMA,
        pltpu.SemaphoreType.DMA,
        pltpu.SemaphoreType.DMA,
        pltpu.SemaphoreType.DMA,
        pltpu.SemaphoreType.DMA,
        pltpu.SemaphoreType.DMA,
    ],
    compiler_params=pltpu.CompilerParams(use_tc_tiling_on_sc=False),
)
def _agg_kernel(y_hbm, row_hbm, col_hbm, attr_hbm, out_hbm,
                row_v, col_v, attr_v, g0, g1, g2, g3, s0, s1,
                acc_sh, gs0, gs1, gs2, gs3, ss0, ss1):
    c = lax.axis_index("c")
    s = lax.axis_index("s")
    tile = c * NS + s

    pltpu.sync_copy(row_hbm.at[tile], row_v)
    pltpu.sync_copy(col_hbm.at[tile], col_v)
    pltpu.sync_copy(attr_hbm.at[tile], attr_v)

    gbufs = (g0, g1, g2, g3)
    gsems = (gs0, gs1, gs2, gs3)
    sbufs = (s0, s1)
    ssems = (ss0, ss1)

    for h in range(2):
        y_half = y_hbm.at[h]

        def zfill(i, _):
            for j in range(DH // L):
                g0[i, pl.ds(j * L, L)] = jnp.zeros((L,), jnp.float32)
            return 0
        lax.fori_loop(0, KB, zfill, 0)

        @pl.when(s < NS - 1)
        def _():
            def zput(i, _):
                pltpu.sync_copy(g0,
                                acc_sh.at[pl.ds(s * CHUNK + i * KB, KB), :])
                return 0
            lax.fori_loop(0, CHUNK // KB, zput, 0)

        @pl.when(s == NS - 1)
        def _():
            def zput(i, _):
                pltpu.sync_copy(
                    g0, acc_sh.at[pl.ds((NS - 1) * CHUNK + i * KB, KB), :])
                return 0
            lax.fori_loop(0, LAST // KB, zput, 0)

        plsc.subcore_barrier()

        def gather_start(b, buf, sem):
            pltpu.make_async_copy(y_half.at[row_v.at[b]], buf, sem).start()

        def gather_wait(b, buf, sem):
            pltpu.make_async_copy(y_half.at[row_v.at[b]], buf, sem).wait()

        def escale(b, gbuf, sbuf):
            def esc(g, _):
                wvec = attr_v[b, pl.ds(g * L, L)]
                for k in range(L):
                    w = wvec[k]
                    e = g * L + k
                    for j in range(DH // L):
                        sbuf[e, pl.ds(j * L, L)] = gbuf[e, pl.ds(j * L, L)] * w
                return 0
            lax.fori_loop(0, KB // L, esc, 0)

        def scatter_start(b, sbuf, sem):
            pltpu.async_copy(sbuf, acc_sh.at[col_v.at[b]], sem, add=True)

        def scatter_wait(b, sbuf, sem):
            pltpu.make_async_copy(sbuf, acc_sh.at[col_v.at[b]], sem).wait()

        for x in range(4):
            gather_start(x, gbufs[x], gsems[x])

        def pipe(j, _):
            for x in range(4):
                b = 4 * j + x
                gather_wait(b, gbufs[x], gsems[x])
                if x >= 2:
                    scatter_wait(b - 2, sbufs[x % 2], ssems[x % 2])
                else:
                    @pl.when(j > 0)
                    def _():
                        scatter_wait(b - 2, sbufs[x % 2], ssems[x % 2])
                escale(b, gbufs[x], sbufs[x % 2])

                @pl.when(b + 4 <= NB - 1)
                def _():
                    gather_start(b + 4, gbufs[x], gsems[x])
                scatter_start(b, sbufs[x % 2], ssems[x % 2])
            return 0
        lax.fori_loop(0, (NB - 1) // 4, pipe, 0)

        gather_wait(NB - 1, g0, gs0)
        scatter_wait(NB - 3, s0, ss0)
        escale(NB - 1, g0, s0)
        scatter_start(NB - 1, s0, ss0)
        scatter_wait(NB - 2, s1, ss1)
        scatter_wait(NB - 1, s0, ss0)

        plsc.subcore_barrier()

        @pl.when(s < NS - 1)
        def _():
            pltpu.sync_copy(acc_sh.at[pl.ds(s * CHUNK, CHUNK), :],
                            out_hbm.at[c, h, pl.ds(s * CHUNK, CHUNK), :])

        @pl.when(s == NS - 1)
        def _():
            pltpu.sync_copy(acc_sh.at[pl.ds((NS - 1) * CHUNK, LAST), :],
                            out_hbm.at[c, h, pl.ds((NS - 1) * CHUNK, LAST), :])


def _ep_body(x_ref, y_ref, p_ref, degt_ref, b_ref, out_ref):
    deg = degt_ref[:, 0:1] + degt_ref[:, 1:2]
    dis = jnp.where(deg > 0, lax.rsqrt(deg), 0.0)
    y_full = jnp.concatenate([y_ref[0], y_ref[1]], axis=1)
    p_full = jnp.concatenate(
        [p_ref[0, 0] + p_ref[1, 0], p_ref[0, 1] + p_ref[1, 1]], axis=1)
    agg = (p_full + y_full) * dis + b_ref[...]
    out_ref[...] = x_ref[...] + jnp.maximum(agg, 0.0)


def _epilogue(x, y, p, deg_t, b):
    blk = 1000
    return pl.pallas_call(
        _ep_body,
        grid=(N // blk,),
        in_specs=[
            pl.BlockSpec((blk, D), lambda i: (i, 0)),
            pl.BlockSpec((2, blk, DH), lambda i: (0, i, 0)),
            pl.BlockSpec((NC, 2, blk, DH), lambda i: (0, 0, i, 0)),
            pl.BlockSpec((blk, NC), lambda i: (i, 0)),
            pl.BlockSpec((1, D), lambda i: (0, 0)),
        ],
        out_specs=pl.BlockSpec((blk, D), lambda i: (i, 0)),
        out_shape=jax.ShapeDtypeStruct((N, D), jnp.float32),
    )(x, y, p, deg_t, b)


def kernel(x, edge_index, edge_attr, W, b):
    row3d = edge_index[0].astype(jnp.int32).reshape(NW, NB, KB)
    col3d = edge_index[1].astype(jnp.int32).reshape(NW, NB, KB)
    attr3d = edge_attr.reshape(NW, NB, KB)

    deg_part = _deg_kernel(col3d, attr3d)
    deg_t = jnp.transpose(deg_part.reshape(NC, N_PAD)[:, :N])
    w_split = jnp.transpose(W.reshape(D, 2, DH), (1, 0, 2))
    y = _matmul_scaled(x, w_split, deg_t)
    p = _agg_kernel(y, row3d, col3d, attr3d)
    return _epilogue(x, y, p, deg_t, b.reshape(1, D))
